# baseline (speedup 1.0000x reference)
import functools

import jax
import jax.numpy as jnp
from jax import lax
from jax.experimental import pallas as pl
from jax.experimental.pallas import tpu as pltpu
from jax.experimental.pallas import tpu_sc as plsc

NC = 2
NS = 16
NW = NC * NS

_MUL_CUR = 36313
_MUL_PRV = 27191


def _make_gather(tok_total, seq, vocab, dim):
    tpw = tok_total // NW
    assert seq % tpw == 0, "worker chunk must evenly divide one sequence"
    n_chunks = tpw // 16
    gchunk = 128
    n_g = tpw // gchunk
    mod = vocab - 1

    mesh = plsc.VectorSubcoreMesh(
        core_axis_name="c", subcore_axis_name="s",
        num_cores=NC, num_subcores=NS)

    @functools.partial(
        pl.kernel, mesh=mesh,
        out_type=jax.ShapeDtypeStruct((tok_total, dim), jnp.float32),
        scratch_types=[
            pltpu.VMEM((tpw + 8,), jnp.int32),
            pltpu.VMEM((n_g, gchunk), jnp.int32),
            pltpu.VMEM((tpw, dim), jnp.float32),
            pltpu.SemaphoreType.DMA,
        ],
        compiler_params=pltpu.CompilerParams(use_tc_tiling_on_sc=False),
    )
    def gather_kernel(tok_hbm, table_hbm, h_hbm, tok_v, idx_v, rows_v, sem):
        wid = lax.axis_index("s") * NC + lax.axis_index("c")
        base = wid * tpw

        @pl.when(wid == 0)
        def _():
            tok_v[pl.ds(0, 16)] = jnp.zeros((16,), jnp.int32)
            pltpu.sync_copy(tok_hbm.at[pl.ds(0, tpw)], tok_v.at[pl.ds(8, tpw)])

        @pl.when(wid != 0)
        def _():
            pltpu.sync_copy(tok_hbm.at[pl.ds(base - 8, tpw + 8)], tok_v)

        lane = lax.iota(jnp.int32, 16)
        lane0 = (16 - lane) >> 4
        s = wid % (seq // tpw)
        seq_start = ((s - 1) >> 31) & 1
        cmod = jnp.int32(mod)
        for i in range(n_chunks):
            cur = tok_v[pl.ds(8 + 16 * i, 16)]
            prv = tok_v[pl.ds(7 + 16 * i, 16)]
            raw = (cur * _MUL_CUR) ^ (prv * _MUL_PRV)
            r = lax.rem(raw, cmod)
            r = r + ((r >> 31) & cmod)
            if i == 0:
                m = lane0 * seq_start
                r = r + (cmod - r) * m
            idx_v[i // 8, pl.ds((i % 8) * 16, 16)] = r

        copies = []
        for j in range(n_g):
            copies.append(pltpu.async_copy(
                table_hbm.at[idx_v.at[j]],
                rows_v.at[pl.ds(j * gchunk, gchunk)], sem))
        for c in copies:
            c.wait()
        pltpu.sync_copy(rows_v, h_hbm.at[pl.ds(base, tpw)])

    return gather_kernel


def _make_matmul(tok_total, dim, model_dim, blk):
    def mm_body(scale_ref, h_ref, w_ref, o_ref):
        acc = lax.dot_general(
            h_ref[...], w_ref[...], (((1,), (1,)), ((), ())),
            preferred_element_type=jnp.float32)
        o_ref[...] = acc * scale_ref[0]

    return pl.pallas_call(
        mm_body,
        grid=(tok_total // blk,),
        in_specs=[
            pl.BlockSpec(memory_space=pltpu.SMEM),
            pl.BlockSpec((blk, dim), lambda i: (i, 0)),
            pl.BlockSpec((model_dim, dim), lambda i: (0, 0)),
        ],
        out_specs=pl.BlockSpec((blk, model_dim), lambda i: (i, 0)),
        out_shape=jax.ShapeDtypeStruct((tok_total, model_dim), jnp.float32),
        compiler_params=pltpu.CompilerParams(
            dimension_semantics=("parallel",)),
    )


def kernel(token_ids, embed_weight, proj_weight, scale):
    batch, seq = token_ids.shape
    vocab, dim = embed_weight.shape
    model_dim = proj_weight.shape[0]
    tok_total = batch * seq

    tok_flat = token_ids.reshape(tok_total)
    h = _make_gather(tok_total, seq, vocab, dim)(tok_flat, embed_weight)
    out = _make_matmul(tok_total, dim, model_dim, 512)(
        scale.reshape(1), h, proj_weight)
    return out.reshape(batch, seq, model_dim)


# speedup vs baseline: 1.6408x; 1.6408x over previous
import functools

import jax
import jax.numpy as jnp
from jax import lax
from jax.experimental import pallas as pl
from jax.experimental.pallas import tpu as pltpu
from jax.experimental.pallas import tpu_sc as plsc

NC = 2
NS = 16
NW = NC * NS

_MUL_CUR = 36313
_MUL_PRV = 27191


def _make_gather(tok_total, seq, vocab, dim):
    tpw = tok_total // NW
    assert seq % tpw == 0, "worker chunk must evenly divide one sequence"
    n_chunks = tpw // 16
    gchunk = 128
    n_g = tpw // gchunk
    mod = vocab - 1

    mesh = plsc.VectorSubcoreMesh(
        core_axis_name="c", subcore_axis_name="s",
        num_cores=NC, num_subcores=NS)

    @functools.partial(
        pl.kernel, mesh=mesh,
        out_type=jax.ShapeDtypeStruct((tok_total, dim), jnp.float32),
        scratch_types=[
            pltpu.VMEM((tpw + 8,), jnp.int32),
            pltpu.VMEM((tpw, dim), jnp.float32),
            pltpu.SemaphoreType.DMA,
        ],
    )
    def gather_kernel(tok_hbm, table_hbm, h_hbm, tok_v, rows_v, sem):
        wid = lax.axis_index("s") * NC + lax.axis_index("c")
        base = wid * tpw

        @pl.when(wid == 0)
        def _():
            tok_v[pl.ds(0, 16)] = jnp.zeros((16,), jnp.int32)
            pltpu.sync_copy(tok_hbm.at[pl.ds(0, tpw)], tok_v.at[pl.ds(8, tpw)])

        @pl.when(wid != 0)
        def _():
            pltpu.sync_copy(tok_hbm.at[pl.ds(base - 8, tpw + 8)], tok_v)

        lane = lax.iota(jnp.int32, 16)
        lane0 = (16 - lane) >> 4
        s = wid % (seq // tpw)
        seq_start = ((s - 1) >> 31) & 1
        cmod = jnp.int32(mod)
        for i in range(n_chunks):
            cur = tok_v[pl.ds(8 + 16 * i, 16)]
            prv = tok_v[pl.ds(7 + 16 * i, 16)]
            raw = (cur * _MUL_CUR) ^ (prv * _MUL_PRV)
            r = lax.rem(raw, cmod)
            r = r + ((r >> 31) & cmod)
            if i == 0:
                m = lane0 * seq_start
                r = r + (cmod - r) * m
            for k in range(16):
                row = 16 * i + k
                pltpu.async_copy(
                    table_hbm.at[pl.ds(r[k], 1)],
                    rows_v.at[pl.ds(row, 1)], sem)
        pltpu.make_async_copy(
            table_hbm.at[pl.ds(0, tpw)], rows_v, sem).wait()
        pltpu.sync_copy(rows_v, h_hbm.at[pl.ds(base, tpw)])

    return gather_kernel


def _make_matmul(tok_total, dim, model_dim, blk):
    def mm_body(scale_ref, h_ref, w_ref, o_ref):
        acc = lax.dot_general(
            h_ref[...], w_ref[...], (((1,), (1,)), ((), ())),
            preferred_element_type=jnp.float32)
        o_ref[...] = acc * scale_ref[0]

    return pl.pallas_call(
        mm_body,
        grid=(tok_total // blk,),
        in_specs=[
            pl.BlockSpec(memory_space=pltpu.SMEM),
            pl.BlockSpec((blk, dim), lambda i: (i, 0)),
            pl.BlockSpec((model_dim, dim), lambda i: (0, 0)),
        ],
        out_specs=pl.BlockSpec((blk, model_dim), lambda i: (i, 0)),
        out_shape=jax.ShapeDtypeStruct((tok_total, model_dim), jnp.float32),
        compiler_params=pltpu.CompilerParams(
            dimension_semantics=("parallel",)),
    )


def kernel(token_ids, embed_weight, proj_weight, scale):
    batch, seq = token_ids.shape
    vocab, dim = embed_weight.shape
    model_dim = proj_weight.shape[0]
    tok_total = batch * seq

    tok_flat = token_ids.reshape(tok_total)
    h = _make_gather(tok_total, seq, vocab, dim)(tok_flat, embed_weight)
    out = _make_matmul(tok_total, dim, model_dim, 512)(
        scale.reshape(1), h, proj_weight)
    return out.reshape(batch, seq, model_dim)


# speedup vs baseline: 2.0335x; 1.2394x over previous
import functools

import jax
import jax.numpy as jnp
from jax import lax
from jax.experimental import pallas as pl
from jax.experimental.pallas import tpu as pltpu
from jax.experimental.pallas import tpu_sc as plsc

NC = 2
NS = 16
NW = NC * NS

_MUL_CUR = 36313
_MUL_PRV = 27191


def _make_untile(vocab, dim):
    lanes = 128
    nfull = vocab // lanes
    nti = -(-vocab // lanes)
    tpwk = -(-nfull // NW)
    chunk = dim * lanes

    mesh = plsc.VectorSubcoreMesh(
        core_axis_name="c", subcore_axis_name="s",
        num_cores=NC, num_subcores=NS)

    nbuf = 8
    nit = -(-tpwk // nbuf)

    @functools.partial(
        pl.kernel, mesh=mesh,
        out_type=jax.ShapeDtypeStruct((nti * dim, lanes), jnp.float32),
        scratch_types=[
            pltpu.VMEM((nbuf, dim, lanes), jnp.float32),
            pltpu.SemaphoreType.DMA,
            pltpu.SemaphoreType.DMA,
        ],
    )
    def untile_kernel(tab_t_hbm, tail_hbm, flat_hbm, buf_v, rsem, wsem):
        wid = lax.axis_index("s") * NC + lax.axis_index("c")
        t0 = wid * tpwk

        def tile_body(i, carry):
            tis = [jnp.minimum(t0 + i * nbuf + b, nfull - 1)
                   for b in range(nbuf)]

            @pl.when(i > 0)
            def _():
                for b in range(nbuf):
                    pltpu.make_async_copy(
                        tab_t_hbm.at[:, pl.ds(0, lanes)], buf_v.at[b],
                        wsem).wait()

            for b in range(nbuf):
                pltpu.async_copy(
                    tab_t_hbm.at[:, pl.ds(tis[b] * lanes, lanes)],
                    buf_v.at[b], rsem)
            for b in range(nbuf):
                pltpu.make_async_copy(
                    tab_t_hbm.at[:, pl.ds(0, lanes)], buf_v.at[b],
                    rsem).wait()
            for b in range(nbuf):
                pltpu.async_copy(
                    buf_v.at[b],
                    flat_hbm.at[pl.ds(tis[b] * dim, dim), :], wsem)
            return carry

        lax.fori_loop(0, nit, tile_body, 0)
        for b in range(nbuf):
            pltpu.make_async_copy(
                tab_t_hbm.at[:, pl.ds(0, lanes)], buf_v.at[b], wsem).wait()

        @pl.when(wid == 0)
        def _():
            pltpu.sync_copy(tail_hbm, buf_v.at[0])
            pltpu.sync_copy(buf_v.at[0],
                            flat_hbm.at[pl.ds(nfull * dim, dim), :])

    return untile_kernel


def _make_gather(tok_total, seq, vocab, dim):
    tpw = tok_total // NW
    assert seq % tpw == 0, "worker chunk must evenly divide one sequence"
    n_chunks = tpw // 16
    mod = vocab - 1

    mesh = plsc.VectorSubcoreMesh(
        core_axis_name="c", subcore_axis_name="s",
        num_cores=NC, num_subcores=NS)

    wave = 32
    n_waves = tpw // wave
    seg = 16

    @functools.partial(
        pl.kernel, mesh=mesh,
        out_type=jax.ShapeDtypeStruct((tok_total * dim,), jnp.float32),
        scratch_types=[
            pltpu.VMEM((tpw + 8,), jnp.int32),
            pltpu.VMEM((wave, dim, seg), jnp.float32),
            pltpu.VMEM((tpw * dim,), jnp.float32),
            pltpu.SemaphoreType.DMA,
        ],
        compiler_params=pltpu.CompilerParams(
            use_tc_tiling_on_sc=False, needs_layout_passes=False),
    )
    def gather_kernel(tok_hbm, tab_t_hbm, h_hbm, tok_v, stage_v, rows_v, sem):
        wid = lax.axis_index("s") * NC + lax.axis_index("c")
        base = wid * tpw

        @pl.when(wid == 0)
        def _():
            tok_v[pl.ds(0, 16)] = jnp.zeros((16,), jnp.int32)
            pltpu.sync_copy(tok_hbm.at[pl.ds(0, tpw)], tok_v.at[pl.ds(8, tpw)])

        @pl.when(wid != 0)
        def _():
            pltpu.sync_copy(tok_hbm.at[pl.ds(base - 8, tpw + 8)], tok_v)

        lane = lax.iota(jnp.int32, 16)
        lane0 = (16 - lane) >> 4
        s = wid % (seq // tpw)
        seq_start = ((s - 1) >> 31) & 1
        cmod = jnp.int32(mod)

        def wave_body(w, carry):
            lanes_sel = []
            for b in range(2):
                i = 2 * w + b
                cur = tok_v[pl.ds(8 + 16 * i, 16)]
                prv = tok_v[pl.ds(7 + 16 * i, 16)]
                raw = (cur * _MUL_CUR) ^ (prv * _MUL_PRV)
                r = lax.rem(raw, cmod)
                r = r + ((r >> 31) & cmod)
                if b == 0:
                    first_chunk = ((i - 1) >> 31) & 1
                    m = lane0 * (seq_start * first_chunk)
                    r = r + (cmod - r) * m
                for k in range(16):
                    rk = r[k]
                    row0 = pl.multiple_of((rk >> 7) * dim, dim)
                    c0 = pl.multiple_of(rk & (127 - (seg - 1)), seg)
                    lanes_sel.append(rk & (seg - 1))
                    pltpu.async_copy(
                        tab_t_hbm.at[pl.ds(row0, dim), pl.ds(c0, seg)],
                        stage_v.at[16 * b + k], sem)
            for t in range(wave):
                pltpu.make_async_copy(
                    tab_t_hbm.at[pl.ds(0, dim), pl.ds(0, seg)],
                    stage_v.at[t], sem).wait()
            for t in range(wave):
                tsp = jnp.full((16,), t, jnp.int32)
                csp = jnp.full((16,), 0, jnp.int32) + lanes_sel[t]
                for g in range(dim // 16):
                    jv = lane + 16 * g
                    vals = plsc.load_gather(stage_v, [tsp, jv, csp])
                    rows_v[pl.ds((wave * w + t) * dim + 16 * g, 16)] = vals
            return carry

        lax.fori_loop(0, n_waves, wave_body, 0)
        pltpu.sync_copy(rows_v, h_hbm.at[pl.ds(base * dim, tpw * dim)])

    return gather_kernel


def _make_matmul(tok_total, dim, model_dim, blk):
    def mm_body(scale_ref, h_ref, w_ref, o_ref):
        acc = lax.dot_general(
            h_ref[...], w_ref[...], (((1,), (1,)), ((), ())),
            preferred_element_type=jnp.float32)
        o_ref[...] = acc * scale_ref[0]

    return pl.pallas_call(
        mm_body,
        grid=(tok_total // blk,),
        in_specs=[
            pl.BlockSpec(memory_space=pltpu.SMEM),
            pl.BlockSpec((blk, dim), lambda i: (i, 0)),
            pl.BlockSpec((model_dim, dim), lambda i: (0, 0)),
        ],
        out_specs=pl.BlockSpec((blk, model_dim), lambda i: (i, 0)),
        out_shape=jax.ShapeDtypeStruct((tok_total, model_dim), jnp.float32),
        compiler_params=pltpu.CompilerParams(
            dimension_semantics=("parallel",)),
    )


def kernel(token_ids, embed_weight, proj_weight, scale):
    batch, seq = token_ids.shape
    vocab, dim = embed_weight.shape
    model_dim = proj_weight.shape[0]
    tok_total = batch * seq

    tok_flat = token_ids.reshape(tok_total)
    table_t = embed_weight.T
    nfull = (vocab // 128) * 128
    tail = jnp.pad(embed_weight[nfull:, :].T, ((0, 0), (0, 128 - (vocab - nfull))))
    table_tiles = _make_untile(vocab, dim)(table_t, tail)
    h_flat = _make_gather(tok_total, seq, vocab, dim)(tok_flat, table_tiles)
    h = h_flat.reshape(tok_total, dim)
    out = _make_matmul(tok_total, dim, model_dim, 512)(
        scale.reshape(1), h, proj_weight)
    return out.reshape(batch, seq, model_dim)


# speedup vs baseline: 2.1450x; 1.0548x over previous
import functools

import jax
import jax.numpy as jnp
from jax import lax
from jax.experimental import pallas as pl
from jax.experimental.pallas import tpu as pltpu
from jax.experimental.pallas import tpu_sc as plsc

NC = 2
NS = 16
NW = NC * NS

_MUL_CUR = 36313
_MUL_PRV = 27191


def _make_untile(vocab, dim):
    lanes = 128
    nfull = vocab // lanes
    nti = -(-vocab // lanes)
    tpwk = -(-nfull // NW)
    chunk = dim * lanes

    mesh = plsc.VectorSubcoreMesh(
        core_axis_name="c", subcore_axis_name="s",
        num_cores=NC, num_subcores=NS)

    nbuf = 8
    nit = -(-tpwk // nbuf)

    @functools.partial(
        pl.kernel, mesh=mesh,
        out_type=jax.ShapeDtypeStruct((nti * dim, lanes), jnp.float32),
        scratch_types=[
            pltpu.VMEM((nbuf, dim, lanes), jnp.float32),
            pltpu.SemaphoreType.DMA,
            pltpu.SemaphoreType.DMA,
        ],
    )
    def untile_kernel(tab_t_hbm, tail_hbm, flat_hbm, buf_v, rsem, wsem):
        wid = lax.axis_index("s") * NC + lax.axis_index("c")
        t0 = wid * tpwk

        def tile_body(i, carry):
            tis = [jnp.minimum(t0 + i * nbuf + b, nfull - 1)
                   for b in range(nbuf)]

            @pl.when(i > 0)
            def _():
                for b in range(nbuf):
                    pltpu.make_async_copy(
                        tab_t_hbm.at[:, pl.ds(0, lanes)], buf_v.at[b],
                        wsem).wait()

            for b in range(nbuf):
                pltpu.async_copy(
                    tab_t_hbm.at[:, pl.ds(tis[b] * lanes, lanes)],
                    buf_v.at[b], rsem)
            for b in range(nbuf):
                pltpu.make_async_copy(
                    tab_t_hbm.at[:, pl.ds(0, lanes)], buf_v.at[b],
                    rsem).wait()
            for b in range(nbuf):
                pltpu.async_copy(
                    buf_v.at[b],
                    flat_hbm.at[pl.ds(tis[b] * dim, dim), :], wsem)
            return carry

        lax.fori_loop(0, nit, tile_body, 0)
        for b in range(nbuf):
            pltpu.make_async_copy(
                tab_t_hbm.at[:, pl.ds(0, lanes)], buf_v.at[b], wsem).wait()

        @pl.when(wid == 0)
        def _():
            pltpu.sync_copy(tail_hbm, buf_v.at[0])
            pltpu.sync_copy(buf_v.at[0],
                            flat_hbm.at[pl.ds(nfull * dim, dim), :])

    return untile_kernel


def _make_gather(tok_total, seq, vocab, dim):
    tpw = tok_total // NW
    assert seq % tpw == 0, "worker chunk must evenly divide one sequence"
    n_chunks = tpw // 16
    mod = vocab - 1

    mesh = plsc.VectorSubcoreMesh(
        core_axis_name="c", subcore_axis_name="s",
        num_cores=NC, num_subcores=NS)

    wave = 32
    n_waves = tpw // wave
    seg = 16

    @functools.partial(
        pl.kernel, mesh=mesh,
        out_type=jax.ShapeDtypeStruct((tok_total * dim,), jnp.float32),
        scratch_types=[
            pltpu.VMEM((tpw + 8,), jnp.int32),
            pltpu.VMEM((wave, dim, seg), jnp.float32),
            pltpu.VMEM((wave, dim, seg), jnp.float32),
            pltpu.VMEM((tpw * dim,), jnp.float32),
            pltpu.SemaphoreType.DMA,
            pltpu.SemaphoreType.DMA,
        ],
        compiler_params=pltpu.CompilerParams(
            use_tc_tiling_on_sc=False, needs_layout_passes=False),
    )
    def gather_kernel(tok_hbm, tab_t_hbm, h_hbm,
                      tok_v, stage_a, stage_b, rows_v, sem_a, sem_b):
        wid = lax.axis_index("s") * NC + lax.axis_index("c")
        base = wid * tpw

        @pl.when(wid == 0)
        def _():
            tok_v[pl.ds(0, 16)] = jnp.zeros((16,), jnp.int32)
            pltpu.sync_copy(tok_hbm.at[pl.ds(0, tpw)], tok_v.at[pl.ds(8, tpw)])

        @pl.when(wid != 0)
        def _():
            pltpu.sync_copy(tok_hbm.at[pl.ds(base - 8, tpw + 8)], tok_v)

        lane = lax.iota(jnp.int32, 16)
        lane0 = (16 - lane) >> 4
        s = wid % (seq // tpw)
        seq_start = ((s - 1) >> 31) & 1
        cmod = jnp.int32(mod)

        def fire_wave(w, stv, sem_):
            cms = []
            for b in range(2):
                i = 2 * w + b
                cur = tok_v[pl.ds(8 + 16 * i, 16)]
                prv = tok_v[pl.ds(7 + 16 * i, 16)]
                raw = (cur * _MUL_CUR) ^ (prv * _MUL_PRV)
                r = lax.rem(raw, cmod)
                r = r + ((r >> 31) & cmod)
                if b == 0:
                    first_chunk = ((i - 1) >> 31) & 1
                    m = lane0 * (seq_start * first_chunk)
                    r = r + (cmod - r) * m
                cms.append(r & (seg - 1))
                for k in range(16):
                    rk = r[k]
                    row0 = pl.multiple_of((rk >> 7) * dim, dim)
                    c0 = pl.multiple_of(rk & (127 - (seg - 1)), seg)
                    pltpu.async_copy(
                        tab_t_hbm.at[pl.ds(row0, dim), pl.ds(c0, seg)],
                        stv.at[16 * b + k], sem_)
            return cms[0], cms[1]

        def drain_wave(stv, sem_):
            for t in range(wave):
                pltpu.make_async_copy(
                    tab_t_hbm.at[pl.ds(0, dim), pl.ds(0, seg)],
                    stv.at[t], sem_).wait()

        def select_wave(w, stv, cm0, cm1):
            for b, cm in ((0, cm0), (1, cm1)):
                for k in range(16):
                    t = 16 * b + k
                    tsp = jnp.full((16,), t, jnp.int32)
                    csp = jnp.full((16,), 0, jnp.int32) + cm[k]
                    for g in range(dim // 16):
                        jv = lane + 16 * g
                        vals = plsc.load_gather(stv, [tsp, jv, csp])
                        rows_v[pl.ds((wave * w + t) * dim + 16 * g, 16)] = vals

        carry0 = fire_wave(jnp.int32(0), stage_a, sem_a)

        def pair_body(p, carry):
            ca0, ca1 = carry
            cb0, cb1 = fire_wave(2 * p + 1, stage_b, sem_b)
            drain_wave(stage_a, sem_a)
            select_wave(2 * p, stage_a, ca0, ca1)
            wn = jnp.minimum(2 * p + 2, n_waves - 1)
            cn0, cn1 = fire_wave(wn, stage_a, sem_a)
            drain_wave(stage_b, sem_b)
            select_wave(2 * p + 1, stage_b, cb0, cb1)
            return (cn0, cn1)

        lax.fori_loop(0, n_waves // 2, pair_body, carry0)
        drain_wave(stage_a, sem_a)
        pltpu.sync_copy(rows_v, h_hbm.at[pl.ds(base * dim, tpw * dim)])

    return gather_kernel


def _make_matmul(tok_total, dim, model_dim, blk):
    def mm_body(scale_ref, h_ref, w_ref, o_ref):
        acc = lax.dot_general(
            h_ref[...], w_ref[...], (((1,), (1,)), ((), ())),
            preferred_element_type=jnp.float32)
        o_ref[...] = acc * scale_ref[0]

    return pl.pallas_call(
        mm_body,
        grid=(tok_total // blk,),
        in_specs=[
            pl.BlockSpec(memory_space=pltpu.SMEM),
            pl.BlockSpec((blk, dim), lambda i: (i, 0)),
            pl.BlockSpec((model_dim, dim), lambda i: (0, 0)),
        ],
        out_specs=pl.BlockSpec((blk, model_dim), lambda i: (i, 0)),
        out_shape=jax.ShapeDtypeStruct((tok_total, model_dim), jnp.float32),
        compiler_params=pltpu.CompilerParams(
            dimension_semantics=("parallel",)),
    )


def kernel(token_ids, embed_weight, proj_weight, scale):
    batch, seq = token_ids.shape
    vocab, dim = embed_weight.shape
    model_dim = proj_weight.shape[0]
    tok_total = batch * seq

    tok_flat = token_ids.reshape(tok_total)
    table_t = embed_weight.T
    nfull = (vocab // 128) * 128
    tail = jnp.pad(embed_weight[nfull:, :].T, ((0, 0), (0, 128 - (vocab - nfull))))
    table_tiles = _make_untile(vocab, dim)(table_t, tail)
    h_flat = _make_gather(tok_total, seq, vocab, dim)(tok_flat, table_tiles)
    h = h_flat.reshape(tok_total, dim)
    out = _make_matmul(tok_total, dim, model_dim, 512)(
        scale.reshape(1), h, proj_weight)
    return out.reshape(batch, seq, model_dim)
